# baseline (speedup 1.0000x reference)
import functools

import jax
import jax.numpy as jnp
from jax import lax
from jax.experimental import pallas as pl
from jax.experimental.pallas import tpu as pltpu
from jax.experimental.pallas import tpu_sc as plsc

_NC = 2
_NS = 16
_NW = _NC * _NS
_GCH = 64


def _make_sc_gather(B, V, D):
  b_per_w = B // _NW
  n_gchunks = b_per_w // _GCH
  mesh = plsc.VectorSubcoreMesh(
      core_axis_name="c", subcore_axis_name="s",
      num_cores=_NC, num_subcores=_NS,
  )

  @functools.partial(
      pl.kernel,
      out_type=jax.ShapeDtypeStruct((B, D), jnp.bfloat16),
      mesh=mesh,
      scratch_types=[
          pltpu.VMEM((b_per_w,), jnp.int32),
          pltpu.VMEM((b_per_w, D), jnp.float32),
          pltpu.VMEM((b_per_w // 2, D), jnp.int32),
          [pltpu.SemaphoreType.DMA] * (b_per_w // _GCH),
          [pltpu.SemaphoreType.DMA] * (b_per_w // _GCH),
      ],
      compiler_params=pltpu.CompilerParams(
          needs_layout_passes=False,
      ),
  )
  def body(idx_hbm, table_hbm, out_hbm, idx_v, rows_v, out_v, gsems, osems):
    wid = lax.axis_index("s") * _NC + lax.axis_index("c")
    base = wid * b_per_w
    out_view = out_hbm.bitcast(jnp.int32)
    pairs_per_chunk = _GCH // 2

    pltpu.sync_copy(idx_hbm.at[pl.ds(base, b_per_w)], idx_v)

    gathers = []
    for c in range(n_gchunks):
      gathers.append(
          pltpu.async_copy(
              table_hbm.at[idx_v.at[pl.ds(c * _GCH, _GCH)]],
              rows_v.at[pl.ds(c * _GCH, _GCH)],
              gsems[c],
          )
      )

    outs = []
    for c in range(n_gchunks):
      with jax.named_scope(f"gwait{c}"):
        gathers[c].wait()
      pair0 = c * pairs_per_chunk

      with jax.named_scope(f"pack{c}"):

        @plsc.parallel_loop(0, pairs_per_chunk, unroll=4)
        def _pair(m, pair0=pair0):
          r = (pair0 + m) * 2
          for j in range(D // 16):
            x0 = rows_v[r, pl.ds(j * 16, 16)]
            x1 = rows_v[r + 1, pl.ds(j * 16, 16)]
            out_v[pair0 + m, pl.ds(j * 16, 16)] = plsc.bitcast(
                plsc.pack(x0, x1, format=plsc.PackFormat.INTERLEAVED),
                jnp.int32,
            )

      outs.append(
          pltpu.async_copy(
              out_v.at[pl.ds(pair0, pairs_per_chunk)],
              out_view.at[
                  pl.ds(pl.multiple_of(base // 2 + pair0, 8), pairs_per_chunk)
              ],
              osems[c],
          )
      )

    with jax.named_scope("owait"):
      for cp in outs:
        cp.wait()

  return body


def kernel(inputs, weights):
  B = inputs.shape[0]
  V, D = weights.shape
  idx = inputs.astype(jnp.int32)
  fn = _make_sc_gather(B, V, D)
  return fn(idx, weights)


# speedup vs baseline: 1.0012x; 1.0012x over previous
import functools

import jax
import jax.numpy as jnp
from jax import lax
from jax.experimental import pallas as pl
from jax.experimental.pallas import tpu as pltpu
from jax.experimental.pallas import tpu_sc as plsc

_NC = 2
_NS = 16
_NW = _NC * _NS
_GCH = 128


def _make_sc_gather(B, V, D):
  b_per_w = B // _NW
  n_gchunks = b_per_w // _GCH
  mesh = plsc.VectorSubcoreMesh(
      core_axis_name="c", subcore_axis_name="s",
      num_cores=_NC, num_subcores=_NS,
  )

  @functools.partial(
      pl.kernel,
      out_type=jax.ShapeDtypeStruct((B, D), jnp.bfloat16),
      mesh=mesh,
      scratch_types=[
          pltpu.VMEM((b_per_w,), jnp.int32),
          pltpu.VMEM((b_per_w, D), jnp.float32),
          pltpu.VMEM((b_per_w // 2, D), jnp.int32),
          [pltpu.SemaphoreType.DMA] * (b_per_w // _GCH),
          pltpu.SemaphoreType.DMA,
      ],
      compiler_params=pltpu.CompilerParams(
          needs_layout_passes=False,
      ),
  )
  def body(idx_hbm, table_hbm, out_hbm, idx_v, rows_v, out_v, gsems, osem):
    wid = lax.axis_index("s") * _NC + lax.axis_index("c")
    base = wid * b_per_w
    out_view = out_hbm.bitcast(jnp.int32)

    pltpu.sync_copy(idx_hbm.at[pl.ds(base, b_per_w)], idx_v)

    gathers = []
    for c in range(n_gchunks):
      gathers.append(
          pltpu.async_copy(
              table_hbm.at[idx_v.at[pl.ds(c * _GCH, _GCH)]],
              rows_v.at[pl.ds(c * _GCH, _GCH)],
              gsems[c],
          )
      )
    for g in gathers:
      g.wait()

    @plsc.parallel_loop(0, b_per_w // 2, unroll=4)
    def _pair(m):
      r = m * 2
      for j in range(D // 16):
        x0 = rows_v[r, pl.ds(j * 16, 16)]
        x1 = rows_v[r + 1, pl.ds(j * 16, 16)]
        out_v[m, pl.ds(j * 16, 16)] = plsc.bitcast(
            plsc.pack(x0, x1, format=plsc.PackFormat.INTERLEAVED), jnp.int32
        )

    pltpu.async_copy(
        out_v,
        out_view.at[pl.ds(pl.multiple_of(base // 2, 8), b_per_w // 2)],
        osem,
    ).wait()

  return body


def kernel(inputs, weights):
  B = inputs.shape[0]
  V, D = weights.shape
  idx = inputs.astype(jnp.int32)
  fn = _make_sc_gather(B, V, D)
  return fn(idx, weights)


# speedup vs baseline: 1.0018x; 1.0006x over previous
import functools

import jax
import jax.numpy as jnp
from jax import lax
from jax.experimental import pallas as pl
from jax.experimental.pallas import tpu as pltpu
from jax.experimental.pallas import tpu_sc as plsc

_NC = 2
_NS = 16
_NW = _NC * _NS
_GCH = 128


def _make_sc_gather(B, V, D):
  b_per_w = B // _NW
  n_gchunks = b_per_w // _GCH
  mesh = plsc.VectorSubcoreMesh(
      core_axis_name="c", subcore_axis_name="s",
      num_cores=_NC, num_subcores=_NS,
  )

  @functools.partial(
      pl.kernel,
      out_type=jax.ShapeDtypeStruct((B, D), jnp.bfloat16),
      mesh=mesh,
      scratch_types=[
          pltpu.VMEM((b_per_w,), jnp.int32),
          pltpu.VMEM((b_per_w, D), jnp.float32),
          pltpu.VMEM((b_per_w // 2, D), jnp.int32),
          [pltpu.SemaphoreType.DMA] * (b_per_w // _GCH),
          pltpu.SemaphoreType.DMA,
      ],
      compiler_params=pltpu.CompilerParams(
          needs_layout_passes=False,
          disable_bounds_checks=True,
          disable_semaphore_checks=True,
      ),
  )
  def body(idx_hbm, table_hbm, out_hbm, idx_v, rows_v, out_v, gsems, osem):
    wid = lax.axis_index("s") * _NC + lax.axis_index("c")
    base = wid * b_per_w
    out_view = out_hbm.bitcast(jnp.int32)

    pltpu.sync_copy(idx_hbm.at[pl.ds(base, b_per_w)], idx_v)

    gathers = []
    for c in range(n_gchunks):
      gathers.append(
          pltpu.async_copy(
              table_hbm.at[idx_v.at[pl.ds(c * _GCH, _GCH)]],
              rows_v.at[pl.ds(c * _GCH, _GCH)],
              gsems[c],
          )
      )
    for g in gathers:
      g.wait()

    @plsc.parallel_loop(0, b_per_w // 2, unroll=4)
    def _pair(m):
      r = m * 2
      for j in range(D // 16):
        x0 = rows_v[r, pl.ds(j * 16, 16)]
        x1 = rows_v[r + 1, pl.ds(j * 16, 16)]
        out_v[m, pl.ds(j * 16, 16)] = plsc.bitcast(
            plsc.pack(x0, x1, format=plsc.PackFormat.INTERLEAVED), jnp.int32
        )

    pltpu.async_copy(
        out_v,
        out_view.at[pl.ds(pl.multiple_of(base // 2, 8), b_per_w // 2)],
        osem,
    ).wait()

  return body


def kernel(inputs, weights):
  B = inputs.shape[0]
  V, D = weights.shape
  idx = inputs.astype(jnp.int32)
  fn = _make_sc_gather(B, V, D)
  return fn(idx, weights)


# speedup vs baseline: 1.0039x; 1.0021x over previous
import functools

import jax
import jax.numpy as jnp
from jax import lax
from jax.experimental import pallas as pl
from jax.experimental.pallas import tpu as pltpu
from jax.experimental.pallas import tpu_sc as plsc

_NC = 2
_NS = 16
_NW = _NC * _NS
_GCH = 128


def _make_sc_gather(B, V, D):
  b_per_w = B // _NW
  n_gchunks = b_per_w // _GCH
  mesh = plsc.VectorSubcoreMesh(
      core_axis_name="c", subcore_axis_name="s",
      num_cores=_NC, num_subcores=_NS,
  )

  @functools.partial(
      pl.kernel,
      out_type=jax.ShapeDtypeStruct((B, D), jnp.bfloat16),
      mesh=mesh,
      scratch_types=[
          pltpu.VMEM((b_per_w,), jnp.int32),
          pltpu.VMEM((b_per_w, D), jnp.float32),
          pltpu.VMEM((b_per_w // 2, D), jnp.int32),
          [pltpu.SemaphoreType.DMA] * (b_per_w // _GCH),
          pltpu.SemaphoreType.DMA,
      ],
      compiler_params=pltpu.CompilerParams(
          needs_layout_passes=False,
      ),
  )
  def body(idx_hbm, table_hbm, out_hbm, idx_v, rows_v, out_v, gsems, osem):
    wid = lax.axis_index("s") * _NC + lax.axis_index("c")
    base = wid * b_per_w
    out_view = out_hbm.bitcast(jnp.int32)

    pltpu.sync_copy(idx_hbm.at[pl.ds(base, b_per_w)], idx_v)

    gathers = []
    for c in range(n_gchunks):
      gathers.append(
          pltpu.async_copy(
              table_hbm.at[idx_v.at[pl.ds(c * _GCH, _GCH)]],
              rows_v.at[pl.ds(c * _GCH, _GCH)],
              gsems[c],
          )
      )
    for g in gathers:
      g.wait()

    @plsc.parallel_loop(0, b_per_w // 2, unroll=4)
    def _pair(m):
      r = m * 2
      for j in range(D // 16):
        x0 = rows_v[r, pl.ds(j * 16, 16)]
        x1 = rows_v[r + 1, pl.ds(j * 16, 16)]
        out_v[m, pl.ds(j * 16, 16)] = plsc.bitcast(
            plsc.pack(x0, x1, format=plsc.PackFormat.INTERLEAVED), jnp.int32
        )

    pltpu.async_copy(
        out_v,
        out_view.at[pl.ds(pl.multiple_of(base // 2, 8), b_per_w // 2)],
        osem,
    ).wait()

  return body


def kernel(inputs, weights):
  B = inputs.shape[0]
  V, D = weights.shape
  idx = inputs.astype(jnp.int32)
  fn = _make_sc_gather(B, V, D)
  return fn(idx, weights)
